# Initial kernel scaffold; baseline (speedup 1.0000x reference)
import jax
import jax.numpy as jnp
from jax.experimental import pallas as pl


def kernel(hidden_states, topk_weights, topk_ids, gate_up_weight, gate_up_scale, down_weight, down_scale):
    raise NotImplementedError("write your pallas kernel here")


# speedup vs baseline: 2.4548x; 2.4548x over previous
import functools

import jax
import jax.numpy as jnp
from jax.experimental import pallas as pl

_NUM_EXPERTS = 16
_TOP_K = 2
_HIDDEN = 2048
_FFN = 1408
_BLOCK = 128
_TOKENS = 32
_NF = _FFN // _BLOCK
_NK = _HIDDEN // _BLOCK


def _stage1(x_ref, wg_ref, wu_ref, s_ref, o_ref):
    f = pl.program_id(1)
    acc_g = jnp.zeros((_TOKENS, _BLOCK), jnp.float32)
    acc_u = jnp.zeros((_TOKENS, _BLOCK), jnp.float32)
    for kb in range(_NK):
        sl = slice(kb * _BLOCK, (kb + 1) * _BLOCK)
        xk = x_ref[:, sl]
        wg = wg_ref[0, :, sl]
        wu = wu_ref[0, :, sl]
        pg = jax.lax.dot_general(xk, wg, (((1,), (1,)), ((), ())),
                                 preferred_element_type=jnp.float32)
        pu = jax.lax.dot_general(xk, wu, (((1,), (1,)), ((), ())),
                                 preferred_element_type=jnp.float32)
        acc_g = acc_g + pg * s_ref[0, f, kb]
        acc_u = acc_u + pu * s_ref[0, f + _NF, kb]
    o_ref[0] = acc_g * jax.nn.sigmoid(acc_g) * acc_u


def _stage2(a_ref, w_ref, s_ref, ids_ref, wts_ref, o_ref):
    d = pl.program_id(0)
    e = pl.program_id(1)
    ids = ids_ref[...]
    wts = wts_ref[...]
    c = jnp.sum(jnp.where(ids == e, wts, 0.0), axis=1, keepdims=True)
    a = a_ref[0] * c
    acc = jnp.zeros((_TOKENS, _BLOCK), jnp.float32)
    for kb in range(_NF):
        sl = slice(kb * _BLOCK, (kb + 1) * _BLOCK)
        ak = a[:, sl]
        wk = w_ref[0, :, sl]
        p = jax.lax.dot_general(ak, wk, (((1,), (1,)), ((), ())),
                                preferred_element_type=jnp.float32)
        acc = acc + p * s_ref[0, d, kb]

    @pl.when(e == 0)
    def _():
        o_ref[...] = acc

    @pl.when(e != 0)
    def _():
        o_ref[...] += acc


@jax.jit
def kernel(hidden_states, topk_weights, topk_ids, gate_up_weight,
           gate_up_scale, down_weight, down_scale):
    act = pl.pallas_call(
        _stage1,
        grid=(_NUM_EXPERTS, _NF),
        in_specs=[
            pl.BlockSpec((_TOKENS, _HIDDEN), lambda e, f: (0, 0)),
            pl.BlockSpec((1, _BLOCK, _HIDDEN), lambda e, f: (e, f, 0)),
            pl.BlockSpec((1, _BLOCK, _HIDDEN), lambda e, f: (e, f + _NF, 0)),
            pl.BlockSpec((1, 2 * _NF, _NK), lambda e, f: (e, 0, 0)),
        ],
        out_specs=pl.BlockSpec((1, _TOKENS, _BLOCK), lambda e, f: (e, 0, f)),
        out_shape=jax.ShapeDtypeStruct((_NUM_EXPERTS, _TOKENS, _FFN),
                                       jnp.float32),
    )(hidden_states, gate_up_weight, gate_up_weight, gate_up_scale)

    out = pl.pallas_call(
        _stage2,
        grid=(_NK, _NUM_EXPERTS),
        in_specs=[
            pl.BlockSpec((1, _TOKENS, _FFN), lambda d, e: (e, 0, 0)),
            pl.BlockSpec((1, _BLOCK, _FFN), lambda d, e: (e, d, 0)),
            pl.BlockSpec((1, _NK, _NF), lambda d, e: (e, 0, 0)),
            pl.BlockSpec((_TOKENS, _TOP_K), lambda d, e: (0, 0)),
            pl.BlockSpec((_TOKENS, _TOP_K), lambda d, e: (0, 0)),
        ],
        out_specs=pl.BlockSpec((_TOKENS, _BLOCK), lambda d, e: (0, d)),
        out_shape=jax.ShapeDtypeStruct((_TOKENS, _HIDDEN), jnp.float32),
    )(act, down_weight, down_scale, topk_ids, topk_weights)
    return out


# speedup vs baseline: 2.6035x; 1.0606x over previous
import functools

import jax
import jax.numpy as jnp
from jax.experimental import pallas as pl
from jax.experimental.pallas import tpu as pltpu

_NUM_EXPERTS = 16
_TOP_K = 2
_HIDDEN = 2048
_FFN = 1408
_BLOCK = 128
_TOKENS = 32
_NF = _FFN // _BLOCK
_NK = _HIDDEN // _BLOCK


def _scale_row(sv, nblk):
    return jax.lax.broadcast_in_dim(sv, (nblk, _BLOCK), (0,)).reshape(
        1, nblk * _BLOCK)


def _stage1(x_ref, wg_ref, wu_ref, s_ref, o_ref):
    f = pl.program_id(1)
    x = x_ref[...]
    sg = _scale_row(s_ref[0, f, :], _NK)
    su = _scale_row(s_ref[0, f + _NF, :], _NK)
    hg = jax.lax.dot_general(x * sg, wg_ref[0], (((1,), (1,)), ((), ())),
                             preferred_element_type=jnp.float32)
    hu = jax.lax.dot_general(x * su, wu_ref[0], (((1,), (1,)), ((), ())),
                             preferred_element_type=jnp.float32)
    o_ref[0] = hg * jax.nn.sigmoid(hg) * hu


def _stage2(a_ref, w_ref, s_ref, ids_ref, wts_ref, o_ref):
    d = pl.program_id(0)
    e = pl.program_id(1)
    ids = ids_ref[...]
    wts = wts_ref[...]
    c = jnp.sum(jnp.where(ids == e, wts, 0.0), axis=1, keepdims=True)
    sr = _scale_row(s_ref[0, d, :], _NF)
    a = a_ref[0] * (c * sr)
    acc = jax.lax.dot_general(a, w_ref[0], (((1,), (1,)), ((), ())),
                              preferred_element_type=jnp.float32)

    @pl.when(e == 0)
    def _():
        o_ref[...] = acc

    @pl.when(e != 0)
    def _():
        o_ref[...] += acc


@jax.jit
def kernel(hidden_states, topk_weights, topk_ids, gate_up_weight,
           gate_up_scale, down_weight, down_scale):
    act = pl.pallas_call(
        _stage1,
        grid=(_NUM_EXPERTS, _NF),
        in_specs=[
            pl.BlockSpec((_TOKENS, _HIDDEN), lambda e, f: (0, 0)),
            pl.BlockSpec((1, _BLOCK, _HIDDEN), lambda e, f: (e, f, 0)),
            pl.BlockSpec((1, _BLOCK, _HIDDEN), lambda e, f: (e, f + _NF, 0)),
            pl.BlockSpec((1, 2 * _NF, _NK), lambda e, f: (e, 0, 0)),
        ],
        out_specs=pl.BlockSpec((1, _TOKENS, _BLOCK), lambda e, f: (e, 0, f)),
        out_shape=jax.ShapeDtypeStruct((_NUM_EXPERTS, _TOKENS, _FFN),
                                       jnp.float32),
        compiler_params=pltpu.CompilerParams(
            dimension_semantics=("parallel", "parallel")),
    )(hidden_states, gate_up_weight, gate_up_weight, gate_up_scale)

    out = pl.pallas_call(
        _stage2,
        grid=(_NK, _NUM_EXPERTS),
        in_specs=[
            pl.BlockSpec((1, _TOKENS, _FFN), lambda d, e: (e, 0, 0)),
            pl.BlockSpec((1, _BLOCK, _FFN), lambda d, e: (e, d, 0)),
            pl.BlockSpec((1, _NK, _NF), lambda d, e: (e, 0, 0)),
            pl.BlockSpec((_TOKENS, _TOP_K), lambda d, e: (0, 0)),
            pl.BlockSpec((_TOKENS, _TOP_K), lambda d, e: (0, 0)),
        ],
        out_specs=pl.BlockSpec((_TOKENS, _BLOCK), lambda d, e: (0, d)),
        out_shape=jax.ShapeDtypeStruct((_TOKENS, _HIDDEN), jnp.float32),
        compiler_params=pltpu.CompilerParams(
            dimension_semantics=("parallel", "arbitrary")),
    )(act, down_weight, down_scale, topk_ids, topk_weights)
    return out


# speedup vs baseline: 4.8260x; 1.8536x over previous
import jax
import jax.numpy as jnp
from jax.experimental import pallas as pl
from jax.experimental.pallas import tpu as pltpu

_NUM_EXPERTS = 16
_TOP_K = 2
_HIDDEN = 2048
_FFN = 1408
_BLOCK = 128
_TOKENS = 32
_NF = _FFN // _BLOCK
_NK = _HIDDEN // _BLOCK


def _scale_row(sv, nblk):
    return jax.lax.broadcast_in_dim(sv, (nblk, _BLOCK), (0,)).reshape(
        1, nblk * _BLOCK)


def _stage1(x_ref, wg_ref, wu_ref, s_ref, o_ref):
    x = x_ref[...]
    for f in range(_NF):
        sl = slice(f * _BLOCK, (f + 1) * _BLOCK)
        sg = _scale_row(s_ref[0, f, :], _NK)
        su = _scale_row(s_ref[0, f + _NF, :], _NK)
        hg = jax.lax.dot_general(x * sg, wg_ref[0, sl, :],
                                 (((1,), (1,)), ((), ())),
                                 preferred_element_type=jnp.float32)
        hu = jax.lax.dot_general(x * su, wu_ref[0, sl, :],
                                 (((1,), (1,)), ((), ())),
                                 preferred_element_type=jnp.float32)
        o_ref[0, :, sl] = hg * jax.nn.sigmoid(hg) * hu


def _stage2(a_ref, w_ref, s_ref, ids_ref, wts_ref, o_ref):
    e = pl.program_id(0)
    ids = ids_ref[...]
    wts = wts_ref[...]
    c = jnp.sum(jnp.where(ids == e, wts, 0.0), axis=1, keepdims=True)
    a = a_ref[0] * c
    for d in range(_NK):
        sl = slice(d * _BLOCK, (d + 1) * _BLOCK)
        sr = _scale_row(s_ref[0, d, :], _NF)
        p = jax.lax.dot_general(a * sr, w_ref[0, sl, :],
                                (((1,), (1,)), ((), ())),
                                preferred_element_type=jnp.float32)

        @pl.when(e == 0)
        def _():
            o_ref[:, sl] = p

        @pl.when(e != 0)
        def _():
            o_ref[:, sl] += p


@jax.jit
def kernel(hidden_states, topk_weights, topk_ids, gate_up_weight,
           gate_up_scale, down_weight, down_scale):
    act = pl.pallas_call(
        _stage1,
        grid=(_NUM_EXPERTS,),
        in_specs=[
            pl.BlockSpec((_TOKENS, _HIDDEN), lambda e: (0, 0)),
            pl.BlockSpec((1, _FFN, _HIDDEN), lambda e: (e, 0, 0)),
            pl.BlockSpec((1, _FFN, _HIDDEN), lambda e: (e, 1, 0)),
            pl.BlockSpec((1, 2 * _NF, _NK), lambda e: (e, 0, 0)),
        ],
        out_specs=pl.BlockSpec((1, _TOKENS, _FFN), lambda e: (e, 0, 0)),
        out_shape=jax.ShapeDtypeStruct((_NUM_EXPERTS, _TOKENS, _FFN),
                                       jnp.float32),
        compiler_params=pltpu.CompilerParams(
            dimension_semantics=("parallel",)),
    )(hidden_states, gate_up_weight, gate_up_weight, gate_up_scale)

    out = pl.pallas_call(
        _stage2,
        grid=(_NUM_EXPERTS,),
        in_specs=[
            pl.BlockSpec((1, _TOKENS, _FFN), lambda e: (e, 0, 0)),
            pl.BlockSpec((1, _HIDDEN, _FFN), lambda e: (e, 0, 0)),
            pl.BlockSpec((1, _NK, _NF), lambda e: (e, 0, 0)),
            pl.BlockSpec((_TOKENS, _TOP_K), lambda e: (0, 0)),
            pl.BlockSpec((_TOKENS, _TOP_K), lambda e: (0, 0)),
        ],
        out_specs=pl.BlockSpec((_TOKENS, _HIDDEN), lambda e: (0, 0)),
        out_shape=jax.ShapeDtypeStruct((_TOKENS, _HIDDEN), jnp.float32),
        compiler_params=pltpu.CompilerParams(
            dimension_semantics=("arbitrary",)),
    )(act, down_weight, down_scale, topk_ids, topk_weights)
    return out


# speedup vs baseline: 4.8620x; 1.0075x over previous
import jax
import jax.numpy as jnp
from jax.experimental import pallas as pl
from jax.experimental.pallas import tpu as pltpu

_NUM_EXPERTS = 16
_TOP_K = 2
_HIDDEN = 2048
_FFN = 1408
_BLOCK = 128
_TOKENS = 32
_NF = _FFN // _BLOCK
_NK = _HIDDEN // _BLOCK


def _scale_row(sv, nblk):
    return jax.lax.broadcast_in_dim(sv, (nblk, _BLOCK), (0,)).reshape(
        1, nblk * _BLOCK)


def _stage1(x_ref, w_ref, s_ref, o_ref):
    x = x_ref[...]
    for f in range(_NF):
        sl = slice(f * _BLOCK, (f + 1) * _BLOCK)
        slu = slice(_FFN + f * _BLOCK, _FFN + (f + 1) * _BLOCK)
        sg = _scale_row(s_ref[0, f, :], _NK)
        su = _scale_row(s_ref[0, f + _NF, :], _NK)
        hg = jax.lax.dot_general(x * sg, w_ref[0, sl, :],
                                 (((1,), (1,)), ((), ())),
                                 preferred_element_type=jnp.float32)
        hu = jax.lax.dot_general(x * su, w_ref[0, slu, :],
                                 (((1,), (1,)), ((), ())),
                                 preferred_element_type=jnp.float32)
        o_ref[0, :, sl] = hg * jax.nn.sigmoid(hg) * hu


def _stage2(a_ref, wlo_ref, whi_ref, s_ref, ids_ref, wts_ref, o_ref):
    e = pl.program_id(0)
    ids = ids_ref[...]
    wts = wts_ref[...]
    c = jnp.sum(jnp.where(ids == e, wts, 0.0), axis=1, keepdims=True)
    a = a_ref[0] * c
    half = _NK // 2
    for d in range(_NK):
        sl = slice(d * _BLOCK, (d + 1) * _BLOCK)
        w_ref = wlo_ref if d < half else whi_ref
        wsl = slice((d % half) * _BLOCK, (d % half + 1) * _BLOCK)
        sr = _scale_row(s_ref[0, d, :], _NF)
        p = jax.lax.dot_general(a * sr, w_ref[0, wsl, :],
                                (((1,), (1,)), ((), ())),
                                preferred_element_type=jnp.float32)

        @pl.when(e == 0)
        def _():
            o_ref[:, sl] = p

        @pl.when(e != 0)
        def _():
            o_ref[:, sl] += p


@jax.jit
def kernel(hidden_states, topk_weights, topk_ids, gate_up_weight,
           gate_up_scale, down_weight, down_scale):
    act = pl.pallas_call(
        _stage1,
        grid=(_NUM_EXPERTS,),
        in_specs=[
            pl.BlockSpec((_TOKENS, _HIDDEN), lambda e: (0, 0)),
            pl.BlockSpec((1, 2 * _FFN, _HIDDEN), lambda e: (e, 0, 0)),
            pl.BlockSpec((1, 2 * _NF, _NK), lambda e: (e, 0, 0)),
        ],
        out_specs=pl.BlockSpec((1, _TOKENS, _FFN), lambda e: (e, 0, 0)),
        out_shape=jax.ShapeDtypeStruct((_NUM_EXPERTS, _TOKENS, _FFN),
                                       jnp.float32),
        compiler_params=pltpu.CompilerParams(
            dimension_semantics=("parallel",)),
    )(hidden_states, gate_up_weight, gate_up_scale)

    out = pl.pallas_call(
        _stage2,
        grid=(_NUM_EXPERTS,),
        in_specs=[
            pl.BlockSpec((1, _TOKENS, _FFN), lambda e: (e, 0, 0)),
            pl.BlockSpec((1, _HIDDEN // 2, _FFN), lambda e: (e, 0, 0)),
            pl.BlockSpec((1, _HIDDEN // 2, _FFN), lambda e: (e, 1, 0)),
            pl.BlockSpec((1, _NK, _NF), lambda e: (e, 0, 0)),
            pl.BlockSpec((_TOKENS, _TOP_K), lambda e: (0, 0)),
            pl.BlockSpec((_TOKENS, _TOP_K), lambda e: (0, 0)),
        ],
        out_specs=pl.BlockSpec((_TOKENS, _HIDDEN), lambda e: (0, 0)),
        out_shape=jax.ShapeDtypeStruct((_TOKENS, _HIDDEN), jnp.float32),
        compiler_params=pltpu.CompilerParams(
            dimension_semantics=("arbitrary",)),
    )(act, down_weight, down_weight, down_scale, topk_ids, topk_weights)
    return out
